# baseline (speedup 1.0000x reference)
import functools

import jax
import jax.numpy as jnp
from jax import lax
from jax.experimental import pallas as pl
from jax.experimental.pallas import tpu as pltpu
from jax.experimental.pallas import tpu_sc as plsc

_NC = 2
_NS = 16
_NW = _NC * _NS

_B = 1024
_H = 50
_D = 1536
_V = 100
_VP = 128

_BPW = _B // _NW
_NBUF = 2
assert _H % _NBUF == 0
_OUTER = _H // _NBUF


def _sum_tables_kernel(a_ref, b_ref, o_ref):
    o_ref[pl.ds(0, _V)] = a_ref[...] + b_ref[...]


def _sum_tables(w_fw, w_bw):
    return pl.pallas_call(
        _sum_tables_kernel,
        out_shape=jax.ShapeDtypeStruct((_VP, _D), jnp.float32),
    )(w_fw, w_bw)


def _gather_body(table_hbm, idx_hbm, out_hbm, table_sp, idx_v, rows, gsems, ssems, isem):
    sid = lax.axis_index("s")
    wid = sid * _NC + lax.axis_index("c")
    base = wid * _BPW

    nrows = _VP // _NS
    r0 = sid * nrows
    pltpu.sync_copy(table_hbm.at[pl.ds(r0, nrows)], rows[0].at[pl.ds(0, nrows)])
    pltpu.sync_copy(rows[0].at[pl.ds(0, nrows)], table_sp.at[pl.ds(r0, nrows)])

    for h in range(_H):
        pltpu.make_async_copy(
            idx_hbm.at[pl.ds(h * _B + base, _BPW)], idx_v.at[h], isem
        ).start()
    for h in range(_H):
        pltpu.make_async_copy(
            idx_hbm.at[pl.ds(0, _BPW)], idx_v.at[h], isem
        ).wait()
    plsc.subcore_barrier()

    def start_gather(h, b):
        for half in range(_BPW // 16):
            vec = idx_v[h, pl.ds(half * 16, 16)]
            for j in range(16):
                pltpu.make_async_copy(
                    table_sp.at[vec[j]], rows[b].at[half * 16 + j], gsems[b]
                ).start()

    def wait_gather(b):
        pltpu.make_async_copy(
            table_sp.at[pl.ds(0, _BPW)], rows[b], gsems[b]
        ).wait()

    def start_store(h, b):
        pltpu.make_async_copy(
            rows[b], out_hbm.at[h, pl.ds(base, _BPW)], ssems[b]
        ).start()

    def wait_store(b):
        pltpu.make_async_copy(
            rows[b], out_hbm.at[0, pl.ds(base, _BPW)], ssems[b]
        ).wait()

    for b in range(_NBUF):
        start_gather(b, b)

    @pl.loop(0, _OUTER)
    def _outer(j):
        h0 = j * _NBUF
        for b in range(_NBUF):
            wait_gather(b)
            start_store(h0 + b, b)
        @pl.when(j + 1 < _OUTER)
        def _refill():
            for b in range(_NBUF):
                wait_store(b)
                start_gather(h0 + _NBUF + b, b)

    for b in range(_NBUF):
        wait_store(b)


@functools.partial(
    pl.kernel,
    out_type=jax.ShapeDtypeStruct((_H, _B, _D), jnp.float32),
    mesh=plsc.VectorSubcoreMesh(
        core_axis_name="c", subcore_axis_name="s", num_cores=_NC, num_subcores=_NS
    ),
    scratch_types=[
        pltpu.VMEM_SHARED((_VP, _D), jnp.float32),
        pltpu.VMEM((_H, _BPW), jnp.int32),
        [pltpu.VMEM((_BPW, _D), jnp.float32) for _ in range(_NBUF)],
        [pltpu.SemaphoreType.DMA for _ in range(_NBUF)],
        [pltpu.SemaphoreType.DMA for _ in range(_NBUF)],
        pltpu.SemaphoreType.DMA,
    ],
)
def _sc_gather(table_hbm, idx_hbm, out_hbm, table_sp, idx_v, rows, gsems, ssems, isem):
    _gather_body(table_hbm, idx_hbm, out_hbm, table_sp, idx_v, rows, gsems, ssems, isem)


def kernel(indices, W_fw, W_bw):
    w_sum = _sum_tables(W_fw, W_bw)
    idx_t = indices.T.reshape(_H * _B)
    out_t = _sc_gather(w_sum, idx_t)
    return jnp.transpose(out_t, (1, 0, 2))


# speedup vs baseline: 1.0056x; 1.0056x over previous
import functools

import jax
import jax.numpy as jnp
from jax import lax
from jax.experimental import pallas as pl
from jax.experimental.pallas import tpu as pltpu
from jax.experimental.pallas import tpu_sc as plsc

_NC = 2
_NS = 16
_NW = _NC * _NS

_B = 1024
_H = 50
_D = 1536
_V = 100
_VP = 128

_BPW = _B // _NW
_NBUF = 2
assert _H % _NBUF == 0
_OUTER = _H // _NBUF


def _sum_tables_kernel(a_ref, b_ref, o_ref):
    o_ref[pl.ds(0, _V)] = a_ref[...] + b_ref[...]


def _sum_tables(w_fw, w_bw):
    return pl.pallas_call(
        _sum_tables_kernel,
        out_shape=jax.ShapeDtypeStruct((_VP, _D), jnp.float32),
    )(w_fw, w_bw)


def _gather_body(table_hbm, idx_hbm, out_hbm, table_sp, idx_v, rows, gsems, ssems, isem):
    sid = lax.axis_index("s")
    wid = sid * _NC + lax.axis_index("c")
    base = wid * _BPW

    for h in range(_H):
        pltpu.make_async_copy(
            idx_hbm.at[pl.ds(h * _B + base, _BPW)], idx_v.at[h], isem
        ).start()

    nrows = _VP // _NS
    r0 = sid * nrows
    pltpu.sync_copy(table_hbm.at[pl.ds(r0, nrows)], rows[0].at[pl.ds(0, nrows)])
    pltpu.sync_copy(rows[0].at[pl.ds(0, nrows)], table_sp.at[pl.ds(r0, nrows)])

    for h in range(_H):
        pltpu.make_async_copy(
            idx_hbm.at[pl.ds(0, _BPW)], idx_v.at[h], isem
        ).wait()
    plsc.subcore_barrier()

    def start_gather(h, b):
        for half in range(_BPW // 16):
            vec = idx_v[h, pl.ds(half * 16, 16)]
            for j in range(16):
                pltpu.make_async_copy(
                    table_sp.at[vec[j]], rows[b].at[half * 16 + j], gsems[b]
                ).start()

    def wait_gather(b):
        pltpu.make_async_copy(
            table_sp.at[pl.ds(0, _BPW)], rows[b], gsems[b]
        ).wait()

    def start_store(h, b):
        pltpu.make_async_copy(
            rows[b], out_hbm.at[h, pl.ds(base, _BPW)], ssems[b]
        ).start()

    def wait_store(b):
        pltpu.make_async_copy(
            rows[b], out_hbm.at[0, pl.ds(base, _BPW)], ssems[b]
        ).wait()

    for b in range(_NBUF):
        start_gather(b, b)

    @pl.loop(0, _OUTER)
    def _outer(j):
        h0 = j * _NBUF
        for b in range(_NBUF):
            wait_gather(b)
            start_store(h0 + b, b)
        @pl.when(j + 1 < _OUTER)
        def _refill():
            for b in range(_NBUF):
                wait_store(b)
                start_gather(h0 + _NBUF + b, b)

    for b in range(_NBUF):
        wait_store(b)


@functools.partial(
    pl.kernel,
    out_type=jax.ShapeDtypeStruct((_H, _B, _D), jnp.float32),
    mesh=plsc.VectorSubcoreMesh(
        core_axis_name="c", subcore_axis_name="s", num_cores=_NC, num_subcores=_NS
    ),
    scratch_types=[
        pltpu.VMEM_SHARED((_VP, _D), jnp.float32),
        pltpu.VMEM((_H, _BPW), jnp.int32),
        [pltpu.VMEM((_BPW, _D), jnp.float32) for _ in range(_NBUF)],
        [pltpu.SemaphoreType.DMA for _ in range(_NBUF)],
        [pltpu.SemaphoreType.DMA for _ in range(_NBUF)],
        pltpu.SemaphoreType.DMA,
    ],
)
def _sc_gather(table_hbm, idx_hbm, out_hbm, table_sp, idx_v, rows, gsems, ssems, isem):
    _gather_body(table_hbm, idx_hbm, out_hbm, table_sp, idx_v, rows, gsems, ssems, isem)


def kernel(indices, W_fw, W_bw):
    w_sum = _sum_tables(W_fw, W_bw)
    idx_t = indices.T.reshape(_H * _B)
    out_t = _sc_gather(w_sum, idx_t)
    return jnp.transpose(out_t, (1, 0, 2))
